# Initial kernel scaffold; baseline (speedup 1.0000x reference)
import functools

import jax
import jax.numpy as jnp
from jax import lax
from jax.experimental import pallas as pl
from jax.experimental.pallas import tpu as pltpu
from jax.experimental.pallas import tpu_sc as plsc

NC, NS = 2, 16
NW = NC * NS
CHUNK = 128
NBUF = 2


def _make_gather(B, D):
    assert B % (NW * CHUNK) == 0
    bpw = B // NW
    nchunks = bpw // CHUNK

    mesh = plsc.VectorSubcoreMesh(
        core_axis_name="c", subcore_axis_name="s",
        num_cores=NC, num_subcores=NS)

    @functools.partial(
        pl.kernel,
        out_type=jax.ShapeDtypeStruct((B, D), jnp.float32),
        mesh=mesh,
        scratch_types=[
            pltpu.VMEM((bpw,), jnp.int32),
            pltpu.VMEM((NBUF, CHUNK, D), jnp.float32),
            pltpu.SemaphoreType.DMA,
            pltpu.SemaphoreType.DMA,
            pltpu.SemaphoreType.DMA,
            pltpu.SemaphoreType.DMA,
        ],
    )
    def k(idx_hbm, table_hbm, out_hbm, idx_v, rows_v, g0, g1, s0, s1):
        gsem = (g0, g1)
        ssem = (s0, s1)
        wid = lax.axis_index("s") * NC + lax.axis_index("c")
        base = wid * bpw
        pltpu.sync_copy(idx_hbm.at[pl.ds(base, bpw)], idx_v)

        def gather_start(c, b):
            pltpu.async_copy(
                table_hbm.at[idx_v.at[pl.ds(c * CHUNK, CHUNK)]],
                rows_v.at[b], gsem[b])

        def gather_wait(b):
            pltpu.make_async_copy(
                table_hbm.at[idx_v.at[pl.ds(0, CHUNK)]],
                rows_v.at[b], gsem[b]).wait()

        def scatter_start(c, b):
            pltpu.async_copy(
                rows_v.at[b],
                out_hbm.at[pl.ds(base + c * CHUNK, CHUNK)], ssem[b])

        def scatter_wait(c, b):
            pltpu.make_async_copy(
                rows_v.at[b],
                out_hbm.at[pl.ds(base + c * CHUNK, CHUNK)], ssem[b]).wait()

        for b in range(NBUF):
            gather_start(b, b)

        def body(j, carry):
            for b in range(NBUF):
                c = j * NBUF + b
                gather_wait(b)
                scatter_start(c, b)
                scatter_wait(c, b)
                gather_start(c + NBUF, b)
            return carry

        lax.fori_loop(0, (nchunks - NBUF) // NBUF, body, 0)

        for b in range(NBUF):
            c = nchunks - NBUF + b
            gather_wait(b)
            scatter_start(c, b)
        for b in range(NBUF):
            c = nchunks - NBUF + b
            scatter_wait(c, b)

    return k


def kernel(x, weight):
    B, L = x.shape
    V, D = weight.shape
    idx = x.reshape(-1).astype(jnp.int32)
    table = jax.lax.bitcast_convert_type(weight, jnp.float32).reshape(V, 2 * D)
    out = _make_gather(B * L, 2 * D)(idx, table)
    return jax.lax.bitcast_convert_type(
        out.reshape(B, L, D, 2), jnp.complex64)


# speedup vs baseline: 1.4328x; 1.4328x over previous
import functools

import jax
import jax.numpy as jnp
from jax import lax
from jax.experimental import pallas as pl
from jax.experimental.pallas import tpu as pltpu
from jax.experimental.pallas import tpu_sc as plsc

NC, NS = 2, 16
NW = NC * NS
CHUNK = 128
NBUF = 2
NPLANE = 2


def _make_gather(B, D):
    assert B % (NW * CHUNK) == 0
    bpw = B // NW
    nchunks = bpw // CHUNK

    mesh = plsc.VectorSubcoreMesh(
        core_axis_name="c", subcore_axis_name="s",
        num_cores=NC, num_subcores=NS)

    f32 = jnp.float32

    @functools.partial(
        pl.kernel,
        out_type=(jax.ShapeDtypeStruct((B, D), f32),
                  jax.ShapeDtypeStruct((B, D), f32)),
        mesh=mesh,
        scratch_types=[
            pltpu.VMEM((bpw,), jnp.int32),
            pltpu.VMEM((NBUF, NPLANE, CHUNK, D), f32),
        ] + [pltpu.SemaphoreType.DMA] * (2 * NBUF * NPLANE),
    )
    def k(idx_hbm, wr_hbm, wi_hbm, outr_hbm, outi_hbm, idx_v, rows_v, *sems):
        tables = (wr_hbm, wi_hbm)
        outs = (outr_hbm, outi_hbm)
        gsem = [sems[0:2], sems[2:4]]
        ssem = [sems[4:6], sems[6:8]]
        wid = lax.axis_index("s") * NC + lax.axis_index("c")
        base = wid * bpw
        pltpu.sync_copy(idx_hbm.at[pl.ds(base, bpw)], idx_v)

        def gather_start(c, b):
            for p in range(NPLANE):
                pltpu.async_copy(
                    tables[p].at[idx_v.at[pl.ds(c * CHUNK, CHUNK)]],
                    rows_v.at[b, p], gsem[b][p])

        def gather_wait(b):
            for p in range(NPLANE):
                pltpu.make_async_copy(
                    tables[p].at[idx_v.at[pl.ds(0, CHUNK)]],
                    rows_v.at[b, p], gsem[b][p]).wait()

        def scatter_start(c, b):
            for p in range(NPLANE):
                pltpu.async_copy(
                    rows_v.at[b, p],
                    outs[p].at[pl.ds(base + c * CHUNK, CHUNK)], ssem[b][p])

        def scatter_wait(c, b):
            for p in range(NPLANE):
                pltpu.make_async_copy(
                    rows_v.at[b, p],
                    outs[p].at[pl.ds(base + c * CHUNK, CHUNK)],
                    ssem[b][p]).wait()

        for b in range(NBUF):
            gather_start(b, b)

        def body(j, carry):
            for b in range(NBUF):
                c = j * NBUF + b
                gather_wait(b)
                scatter_start(c, b)
                scatter_wait(c, b)
                gather_start(c + NBUF, b)
            return carry

        lax.fori_loop(0, (nchunks - NBUF) // NBUF, body, 0)

        for b in range(NBUF):
            c = nchunks - NBUF + b
            gather_wait(b)
            scatter_start(c, b)
        for b in range(NBUF):
            c = nchunks - NBUF + b
            scatter_wait(c, b)

    return k


def kernel(x, weight):
    B, L = x.shape
    V, D = weight.shape
    idx = x.reshape(-1).astype(jnp.int32)
    wr = jnp.real(weight)
    wi = jnp.imag(weight)
    outr, outi = _make_gather(B * L, D)(idx, wr, wi)
    return lax.complex(outr, outi).reshape(B, L, D)


# speedup vs baseline: 1.7581x; 1.2271x over previous
import functools

import jax
import jax.numpy as jnp
from jax import lax
from jax.experimental import pallas as pl
from jax.experimental.pallas import tpu as pltpu
from jax.experimental.pallas import tpu_sc as plsc

NC, NS = 2, 16
NW = NC * NS
CHUNK = 128
NBUF = 2
NPLANE = 2


def _make_gather(B, D):
    assert B % (NW * CHUNK) == 0
    bpw = B // NW
    nchunks = bpw // CHUNK

    mesh = plsc.VectorSubcoreMesh(
        core_axis_name="c", subcore_axis_name="s",
        num_cores=NC, num_subcores=NS)

    f32 = jnp.float32

    @functools.partial(
        pl.kernel,
        out_type=(jax.ShapeDtypeStruct((B, D), f32),
                  jax.ShapeDtypeStruct((B, D), f32)),
        mesh=mesh,
        scratch_types=[
            pltpu.VMEM((bpw,), jnp.int32),
            pltpu.VMEM((NBUF, NPLANE, CHUNK, D), f32),
        ] + [pltpu.SemaphoreType.DMA] * (2 * NBUF * NPLANE),
    )
    def k(idx_hbm, wr_hbm, wi_hbm, outr_hbm, outi_hbm, idx_v, rows_v, *sems):
        tables = (wr_hbm, wi_hbm)
        outs = (outr_hbm, outi_hbm)
        gsem = [sems[0:2], sems[2:4]]
        ssem = [sems[4:6], sems[6:8]]
        wid = lax.axis_index("s") * NC + lax.axis_index("c")
        base = wid * bpw
        pltpu.sync_copy(idx_hbm.at[pl.ds(base, bpw)], idx_v)

        def gather_start(c, b):
            for p in range(NPLANE):
                pltpu.async_copy(
                    tables[p].at[idx_v.at[pl.ds(c * CHUNK, CHUNK)]],
                    rows_v.at[b, p], gsem[b][p])

        def gather_wait(b):
            for p in range(NPLANE):
                pltpu.make_async_copy(
                    tables[p].at[idx_v.at[pl.ds(0, CHUNK)]],
                    rows_v.at[b, p], gsem[b][p]).wait()

        def scatter_start(c, b):
            for p in range(NPLANE):
                pltpu.async_copy(
                    rows_v.at[b, p],
                    outs[p].at[pl.ds(base + c * CHUNK, CHUNK)], ssem[b][p])

        def scatter_wait(c, b):
            for p in range(NPLANE):
                pltpu.make_async_copy(
                    rows_v.at[b, p],
                    outs[p].at[pl.ds(base + c * CHUNK, CHUNK)],
                    ssem[b][p]).wait()

        for b in range(NBUF):
            gather_start(b, b)

        def body(j, carry):
            for b in range(NBUF):
                c = j * NBUF + b
                gather_wait(b)
                scatter_start(c, b)
                scatter_wait(c, b)
                gather_start(c + NBUF, b)
            return carry

        lax.fori_loop(0, (nchunks - NBUF) // NBUF, body, 0)

        for b in range(NBUF):
            c = nchunks - NBUF + b
            gather_wait(b)
            scatter_start(c, b)
        for b in range(NBUF):
            c = nchunks - NBUF + b
            scatter_wait(c, b)

    return k


def kernel(x, weight):
    B, L = x.shape
    V, D = weight.shape
    idx = x.T.reshape(-1).astype(jnp.int32)
    wr = jnp.real(weight)
    wi = jnp.imag(weight)
    outr, outi = _make_gather(B * L, D)(idx, wr, wi)
    out = lax.complex(outr.reshape(L, B, D), outi.reshape(L, B, D))
    return jnp.transpose(out, (1, 0, 2))


# speedup vs baseline: 1.7827x; 1.0140x over previous
import functools

import jax
import jax.numpy as jnp
from jax import lax
from jax.experimental import pallas as pl
from jax.experimental.pallas import tpu as pltpu
from jax.experimental.pallas import tpu_sc as plsc

NC, NS = 2, 16
NW = NC * NS
CHUNK = 128
NBUF = 2


def _make_gather_plane(B, D):
    assert B % (NW * CHUNK) == 0
    bpw = B // NW
    nchunks = bpw // CHUNK

    mesh = plsc.VectorSubcoreMesh(
        core_axis_name="c", subcore_axis_name="s",
        num_cores=NC, num_subcores=NS)

    @functools.partial(
        pl.kernel,
        out_type=jax.ShapeDtypeStruct((B, D), jnp.float32),
        mesh=mesh,
        scratch_types=[
            pltpu.VMEM((bpw,), jnp.int32),
            pltpu.VMEM((NBUF, CHUNK, D), jnp.float32),
        ] + [pltpu.SemaphoreType.DMA] * (2 * NBUF),
    )
    def k(idx_hbm, w_hbm, out_hbm, idx_v, rows_v, *sems):
        gsem = sems[0:NBUF]
        ssem = sems[NBUF:2 * NBUF]
        wid = lax.axis_index("s") * NC + lax.axis_index("c")
        base = wid * bpw
        pltpu.sync_copy(idx_hbm.at[pl.ds(base, bpw)], idx_v)

        def gather_start(c, b):
            pltpu.async_copy(
                w_hbm.at[idx_v.at[pl.ds(c * CHUNK, CHUNK)]],
                rows_v.at[b], gsem[b])

        def gather_wait(b):
            pltpu.make_async_copy(
                w_hbm.at[idx_v.at[pl.ds(0, CHUNK)]],
                rows_v.at[b], gsem[b]).wait()

        def scatter_start(c, b):
            pltpu.async_copy(
                rows_v.at[b],
                out_hbm.at[pl.ds(base + c * CHUNK, CHUNK)], ssem[b])

        def scatter_wait(c, b):
            pltpu.make_async_copy(
                rows_v.at[b],
                out_hbm.at[pl.ds(base + c * CHUNK, CHUNK)], ssem[b]).wait()

        for b in range(NBUF):
            gather_start(b, b)

        def body(j, carry):
            for b in range(NBUF):
                c = j * NBUF + b
                gather_wait(b)
                scatter_start(c, b)
                scatter_wait(c, b)
                gather_start(c + NBUF, b)
            return carry

        lax.fori_loop(0, (nchunks - NBUF) // NBUF, body, 0)

        for b in range(NBUF):
            c = nchunks - NBUF + b
            gather_wait(b)
            scatter_start(c, b)
        for b in range(NBUF):
            c = nchunks - NBUF + b
            scatter_wait(c, b)

    return k


def kernel(x, weight):
    B, L = x.shape
    V, D = weight.shape
    idx = x.T.reshape(-1).astype(jnp.int32)
    gather = _make_gather_plane(B * L, D)
    outr = gather(idx, jnp.real(weight))
    outi = gather(idx, jnp.imag(weight))
    out = lax.complex(outr.reshape(L, B, D), outi.reshape(L, B, D))
    return jnp.transpose(out, (1, 0, 2))
